# baseline (speedup 1.0000x reference)
import numpy as np
import jax
import jax.numpy as jnp
from jax.experimental import pallas as pl
from jax.experimental.pallas import tpu as pltpu

_LANE = 128


def _make_kernel(H, W, HW, C, K, pad, n_per, n_rows, n_hw):
    n_tiles = C // _LANE
    HWT = HW // n_hw

    def _body(x_ref, w_ref, rc_ref, o_ref, acc_ref):
        j = pl.program_id(1)
        h = pl.program_id(2)

        xb = x_ref[0]
        ps = xb[:, 0:_LANE].astype(jnp.float32)
        pm = ps
        for t in range(1, n_tiles):
            xt = xb[:, t * _LANE:(t + 1) * _LANE].astype(jnp.float32)
            ps = ps + xt
            pm = jnp.maximum(pm, xt)

        ts = jnp.swapaxes(ps, 0, 1)
        tm = jnp.swapaxes(pm, 0, 1)
        ss = ts.reshape(_LANE // 8, 8, HWT)
        sm = tm.reshape(_LANE // 8, 8, HWT)
        s8 = jnp.sum(ss, axis=0)
        m8 = jnp.max(sm, axis=0)
        acc_ref[pl.ds(j, 1), pl.ds(h * HWT, HWT)] = \
            jnp.sum(s8, axis=0, keepdims=True)
        acc_ref[pl.ds(n_per + j, 1), pl.ds(h * HWT, HWT)] = \
            jnp.max(m8, axis=0, keepdims=True)

        @pl.when((j == n_per - 1) & (h == n_hw - 1))
        def _epilogue():
            two = acc_ref[...]
            row = rc_ref[0:1, :]
            col = rc_ref[1:2, :]

            s_dx = [jnp.zeros((n_rows, HW), jnp.float32) for _ in range(K)]
            for dy in range(K):
                shift = (-(dy - pad) * W) % HW
                r = two if shift == 0 else pltpu.roll(two, shift=shift, axis=1)
                rm = jnp.where((row >= pad - dy) & (row < H + pad - dy), r, 0.0)
                for dx in range(K):
                    w2 = w_ref[:, dy * K + dx:dy * K + dx + 1]
                    s_dx[dx] = s_dx[dx] + w2 * rm

            acc = jnp.zeros((n_rows, HW), jnp.float32)
            for dx in range(K):
                shift = (-(dx - pad)) % HW
                t = s_dx[dx] if shift == 0 else pltpu.roll(s_dx[dx],
                                                           shift=shift, axis=1)
                acc = acc + jnp.where((col >= pad - dx) & (col < W + pad - dx),
                                      t, 0.0)
            conv = acc[0:n_per, :] + acc[n_per:2 * n_per, :]
            o_ref[0] = jax.nn.sigmoid(conv).astype(o_ref.dtype)

    return _body


def kernel(x, weight):
    N, C, H, W = x.shape
    K = weight.shape[-1]
    pad = K // 2
    HW = H * W

    n_cores = 2 if N % 2 == 0 else 1
    n_per = N // n_cores
    n_rows = 2 * n_per
    n_hw = 4 if HW % (4 * 8) == 0 else 1

    wt = weight.astype(jnp.float32).reshape(2, K * K)
    w_rows = jnp.concatenate(
        [jnp.broadcast_to(wt[0:1] / C, (n_per, K * K)),
         jnp.broadcast_to(wt[1:2], (n_per, K * K))], axis=0)

    rows = np.repeat(np.arange(H, dtype=np.int32), W)
    cols = np.tile(np.arange(W, dtype=np.int32), H)
    rc = jnp.asarray(np.stack([rows, cols], axis=0))

    xt = jnp.transpose(x, (0, 2, 3, 1)).reshape(N, HW, C)

    kernel_fn = _make_kernel(H, W, HW, C, K, pad, n_per, n_rows, n_hw)
    out = pl.pallas_call(
        kernel_fn,
        out_shape=jax.ShapeDtypeStruct((n_cores, n_per, HW), x.dtype),
        grid=(n_cores, n_per, n_hw),
        in_specs=[
            pl.BlockSpec((1, HW // n_hw, C),
                         lambda i, j, h: (i * n_per + j, h, 0)),
            pl.BlockSpec((n_rows, K * K), lambda i, j, h: (0, 0)),
            pl.BlockSpec((2, HW), lambda i, j, h: (0, 0)),
        ],
        out_specs=pl.BlockSpec((1, n_per, HW), lambda i, j, h: (i, 0, 0)),
        scratch_shapes=[pltpu.VMEM((n_rows, HW), jnp.float32)],
        compiler_params=pltpu.CompilerParams(
            dimension_semantics=("parallel", "arbitrary", "arbitrary"),
            vmem_limit_bytes=48 * 1024 * 1024),
    )(xt, w_rows, rc)
    return out.reshape(N, 1, H, W)


# speedup vs baseline: 1.9068x; 1.9068x over previous
import numpy as np
import jax
import jax.numpy as jnp
from jax.experimental import pallas as pl
from jax.experimental.pallas import tpu as pltpu

_LANE = 128


def _make_kernel(H, W, HW, C, K, pad, n_per, n_rows, n_img):
    n_tiles = C // _LANE

    def _body(x_ref, w_ref, rc_ref, o_ref, acc_ref):
        j = pl.program_id(1)

        for v in range(n_img):
            xb = x_ref[v]
            ps = xb[:, 0:_LANE].astype(jnp.float32)
            pm = ps
            for t in range(1, n_tiles):
                xt = xb[:, t * _LANE:(t + 1) * _LANE].astype(jnp.float32)
                ps = ps + xt
                pm = jnp.maximum(pm, xt)

            ts = jnp.swapaxes(ps, 0, 1)
            tm = jnp.swapaxes(pm, 0, 1)
            ss = ts.reshape(_LANE // 8, 8, HW)
            sm = tm.reshape(_LANE // 8, 8, HW)
            s8 = jnp.sum(ss, axis=0)
            m8 = jnp.max(sm, axis=0)
            r = j * n_img + v
            acc_ref[pl.ds(r, 1), :] = jnp.sum(s8, axis=0, keepdims=True)
            acc_ref[pl.ds(n_per + r, 1), :] = jnp.max(m8, axis=0,
                                                      keepdims=True)

        @pl.when(j == pl.num_programs(1) - 1)
        def _epilogue():
            two = acc_ref[...]
            row = rc_ref[0:1, :]
            col = rc_ref[1:2, :]

            s_dx = [jnp.zeros((n_rows, HW), jnp.float32) for _ in range(K)]
            for dy in range(K):
                shift = (-(dy - pad) * W) % HW
                r = two if shift == 0 else pltpu.roll(two, shift=shift, axis=1)
                rm = jnp.where((row >= pad - dy) & (row < H + pad - dy), r, 0.0)
                for dx in range(K):
                    w2 = w_ref[:, dy * K + dx:dy * K + dx + 1]
                    s_dx[dx] = s_dx[dx] + w2 * rm

            acc = jnp.zeros((n_rows, HW), jnp.float32)
            for dx in range(K):
                shift = (-(dx - pad)) % HW
                t = s_dx[dx] if shift == 0 else pltpu.roll(s_dx[dx],
                                                           shift=shift, axis=1)
                acc = acc + jnp.where((col >= pad - dx) & (col < W + pad - dx),
                                      t, 0.0)
            conv = acc[0:n_per, :] + acc[n_per:2 * n_per, :]
            o_ref[0] = jax.nn.sigmoid(conv).astype(o_ref.dtype)

    return _body


def kernel(x, weight):
    N, C, H, W = x.shape
    K = weight.shape[-1]
    pad = K // 2
    HW = H * W

    n_cores = 2 if N % 2 == 0 else 1
    n_per = N // n_cores
    n_rows = 2 * n_per
    n_img = 2 if n_per % 2 == 0 else 1

    wt = weight.astype(jnp.float32).reshape(2, K * K)
    w_rows = jnp.concatenate(
        [jnp.broadcast_to(wt[0:1] / C, (n_per, K * K)),
         jnp.broadcast_to(wt[1:2], (n_per, K * K))], axis=0)

    rows = np.repeat(np.arange(H, dtype=np.int32), W)
    cols = np.tile(np.arange(W, dtype=np.int32), H)
    rc = jnp.asarray(np.stack([rows, cols], axis=0))

    xt = jnp.transpose(x, (0, 2, 3, 1)).reshape(N, HW, C)

    kernel_fn = _make_kernel(H, W, HW, C, K, pad, n_per, n_rows, n_img)
    out = pl.pallas_call(
        kernel_fn,
        out_shape=jax.ShapeDtypeStruct((n_cores, n_per, HW), x.dtype),
        grid=(n_cores, n_per // n_img),
        in_specs=[
            pl.BlockSpec((n_img, HW, C),
                         lambda i, j: ((i * n_per) // n_img + j, 0, 0)),
            pl.BlockSpec((n_rows, K * K), lambda i, j: (0, 0)),
            pl.BlockSpec((2, HW), lambda i, j: (0, 0)),
        ],
        out_specs=pl.BlockSpec((1, n_per, HW), lambda i, j: (i, 0, 0)),
        scratch_shapes=[pltpu.VMEM((n_rows, HW), jnp.float32)],
        compiler_params=pltpu.CompilerParams(
            dimension_semantics=("parallel", "arbitrary"),
            vmem_limit_bytes=48 * 1024 * 1024),
    )(xt, w_rows, rc)
    return out.reshape(N, 1, H, W)


# speedup vs baseline: 1.9312x; 1.0128x over previous
import numpy as np
import jax
import jax.numpy as jnp
from jax.experimental import pallas as pl
from jax.experimental.pallas import tpu as pltpu

_LANE = 128


def _make_kernel(H, W, HW, C, K, pad, n_per, n_rows, n_img):
    n_tiles = C // _LANE

    def _body(x_ref, w_ref, rc_ref, o_ref, acc_ref):
        j = pl.program_id(1)

        for v in range(n_img):
            xb = x_ref[v]
            ps = xb[:, 0:_LANE].astype(jnp.float32)
            pm = ps
            for t in range(1, n_tiles):
                xt = xb[:, t * _LANE:(t + 1) * _LANE].astype(jnp.float32)
                ps = ps + xt
                pm = jnp.maximum(pm, xt)

            ts = jnp.swapaxes(ps, 0, 1)
            tm = jnp.swapaxes(pm, 0, 1)
            ss = ts.reshape(_LANE // 8, 8, HW)
            sm = tm.reshape(_LANE // 8, 8, HW)
            s8 = jnp.sum(ss, axis=0)
            m8 = jnp.max(sm, axis=0)
            r = j * n_img + v
            acc_ref[pl.ds(r, 1), :] = jnp.sum(s8, axis=0, keepdims=True)
            acc_ref[pl.ds(n_per + r, 1), :] = jnp.max(m8, axis=0,
                                                      keepdims=True)

        @pl.when(j == pl.num_programs(1) - 1)
        def _epilogue():
            two = acc_ref[...]
            row = rc_ref[0:1, :]
            col = rc_ref[1:2, :]

            s_dx = [jnp.zeros((n_rows, HW), jnp.float32) for _ in range(K)]
            for dy in range(K):
                shift = (-(dy - pad) * W) % HW
                r = two if shift == 0 else pltpu.roll(two, shift=shift, axis=1)
                rm = jnp.where((row >= pad - dy) & (row < H + pad - dy), r, 0.0)
                for dx in range(K):
                    w2 = w_ref[:, dy * K + dx:dy * K + dx + 1]
                    s_dx[dx] = s_dx[dx] + w2 * rm

            acc = jnp.zeros((n_rows, HW), jnp.float32)
            for dx in range(K):
                shift = (-(dx - pad)) % HW
                t = s_dx[dx] if shift == 0 else pltpu.roll(s_dx[dx],
                                                           shift=shift, axis=1)
                acc = acc + jnp.where((col >= pad - dx) & (col < W + pad - dx),
                                      t, 0.0)
            conv = acc[0:n_per, :] + acc[n_per:2 * n_per, :]
            o_ref[0] = jax.nn.sigmoid(conv).astype(o_ref.dtype)

    return _body


def kernel(x, weight):
    N, C, H, W = x.shape
    K = weight.shape[-1]
    pad = K // 2
    HW = H * W

    n_cores = 2 if N % 2 == 0 else 1
    n_per = N // n_cores
    n_rows = 2 * n_per
    n_img = 4 if n_per % 4 == 0 else (2 if n_per % 2 == 0 else 1)

    wt = weight.astype(jnp.float32).reshape(2, K * K)
    w_rows = jnp.concatenate(
        [jnp.broadcast_to(wt[0:1] / C, (n_per, K * K)),
         jnp.broadcast_to(wt[1:2], (n_per, K * K))], axis=0)

    rows = np.repeat(np.arange(H, dtype=np.int32), W)
    cols = np.tile(np.arange(W, dtype=np.int32), H)
    rc = jnp.asarray(np.stack([rows, cols], axis=0))

    xt = jnp.transpose(x, (0, 2, 3, 1)).reshape(N, HW, C)

    kernel_fn = _make_kernel(H, W, HW, C, K, pad, n_per, n_rows, n_img)
    out = pl.pallas_call(
        kernel_fn,
        out_shape=jax.ShapeDtypeStruct((n_cores, n_per, HW), x.dtype),
        grid=(n_cores, n_per // n_img),
        in_specs=[
            pl.BlockSpec((n_img, HW, C),
                         lambda i, j: ((i * n_per) // n_img + j, 0, 0)),
            pl.BlockSpec((n_rows, K * K), lambda i, j: (0, 0)),
            pl.BlockSpec((2, HW), lambda i, j: (0, 0)),
        ],
        out_specs=pl.BlockSpec((1, n_per, HW), lambda i, j: (i, 0, 0)),
        scratch_shapes=[pltpu.VMEM((n_rows, HW), jnp.float32)],
        compiler_params=pltpu.CompilerParams(
            dimension_semantics=("parallel", "arbitrary"),
            vmem_limit_bytes=48 * 1024 * 1024),
    )(xt, w_rows, rc)
    return out.reshape(N, 1, H, W)
